# Initial kernel scaffold; baseline (speedup 1.0000x reference)
import jax
import jax.numpy as jnp
from jax.experimental import pallas as pl


def kernel(input_ids, word_embeddings, position_embeddings, token_type_embeddings, ln_gamma, ln_beta):
    raise NotImplementedError("write your pallas kernel here")


# speedup vs baseline: 1.8863x; 1.8863x over previous
import functools

import jax
import jax.numpy as jnp
from jax import lax
from jax.experimental import pallas as pl
from jax.experimental.pallas import tpu as pltpu
from jax.experimental.pallas import tpu_sc as plsc

_EPS = 1e-12
_CHUNK = 128


@functools.cache
def _make_sc_gather(n_idx_rows: int, emb: int):
    info = plsc.get_sparse_core_info()
    nw = info.num_cores * info.num_subcores
    rows_per_w = n_idx_rows * _CHUNK // nw
    chunks_per_w = rows_per_w // _CHUNK
    mesh = plsc.VectorSubcoreMesh(core_axis_name="c", subcore_axis_name="s")

    @functools.partial(
        pl.kernel,
        mesh=mesh,
        out_type=jax.ShapeDtypeStruct((n_idx_rows * _CHUNK, emb), jnp.float32),
        scratch_types=[
            pltpu.VMEM((chunks_per_w, _CHUNK), jnp.int32),
            pltpu.VMEM((rows_per_w, emb), jnp.float32),
            pltpu.SemaphoreType.DMA,
        ],
    )
    def gather(idx_hbm, table_hbm, out_hbm, idx_v, rows_v, sem):
        wid = lax.axis_index("s") * info.num_cores + lax.axis_index("c")
        pltpu.sync_copy(idx_hbm.at[pl.ds(wid * chunks_per_w, chunks_per_w)], idx_v)
        copies = [
            pltpu.async_copy(
                table_hbm.at[idx_v.at[j]],
                rows_v.at[pl.ds(j * _CHUNK, _CHUNK)],
                sem,
            )
            for j in range(chunks_per_w)
        ]
        for cp in copies:
            cp.wait()
        pltpu.sync_copy(rows_v, out_hbm.at[pl.ds(wid * rows_per_w, rows_per_w)])

    return gather


def _ln_body(rows_ref, pos_ref, tok_ref, g_ref, b_ref, out_ref):
    x = rows_ref[...] + pos_ref[...] + tok_ref[0:1, :]
    mean = jnp.mean(x, axis=-1, keepdims=True)
    xc = x - mean
    var = jnp.mean(xc * xc, axis=-1, keepdims=True)
    inv = lax.rsqrt(var + _EPS)
    out_ref[...] = xc * inv * g_ref[...] + b_ref[...]


def kernel(input_ids, word_embeddings, position_embeddings, token_type_embeddings, ln_gamma, ln_beta):
    batch, seq = input_ids.shape
    vocab, emb = word_embeddings.shape
    total = batch * seq

    ids = input_ids.astype(jnp.int32).reshape(total // _CHUNK, _CHUNK)
    rows = _make_sc_gather(total // _CHUNK, emb)(ids, word_embeddings)

    blk = 1024
    n_blk = total // blk
    pos_blks = seq // blk
    out = pl.pallas_call(
        _ln_body,
        grid=(n_blk,),
        in_specs=[
            pl.BlockSpec((blk, emb), lambda i: (i, 0)),
            pl.BlockSpec((blk, emb), lambda i: (i % pos_blks, 0)),
            pl.BlockSpec((2, emb), lambda i: (0, 0)),
            pl.BlockSpec((1, emb), lambda i: (0, 0)),
            pl.BlockSpec((1, emb), lambda i: (0, 0)),
        ],
        out_specs=pl.BlockSpec((blk, emb), lambda i: (i, 0)),
        out_shape=jax.ShapeDtypeStruct((total, emb), jnp.float32),
    )(
        rows,
        position_embeddings,
        token_type_embeddings,
        ln_gamma.reshape(1, emb),
        ln_beta.reshape(1, emb),
    )
    return out.reshape(batch, seq, emb)
